# baseline (speedup 1.0000x reference)
import functools
import math

import jax
import jax.numpy as jnp
from jax import lax
from jax.experimental import pallas as pl
from jax.experimental.pallas import tpu as pltpu
from jax.experimental.pallas import tpu_sc as plsc

D_MODEL = 64
SCALE = math.sqrt(D_MODEL)

NW = 32
CH = 128


def _make_kernel(B, V):
    b_per_w = B // NW
    n_ch = b_per_w // CH

    mesh = plsc.VectorSubcoreMesh(core_axis_name="c", subcore_axis_name="s")

    @functools.partial(
        pl.kernel,
        mesh=mesh,
        out_type=jax.ShapeDtypeStruct((B, D_MODEL), jnp.float32),
        scratch_types=[
            pltpu.VMEM((n_ch, CH), jnp.int32),
            pltpu.VMEM((CH, D_MODEL), jnp.float32),
            pltpu.SemaphoreType.DMA,
        ],
        compiler_params=pltpu.CompilerParams(use_tc_tiling_on_sc=False),
    )
    def emb_kernel(idx_hbm, lut_hbm, out_hbm, idx_v, buf, gsem):
        wid = lax.axis_index("s") * 2 + lax.axis_index("c")
        base = wid * b_per_w
        pltpu.sync_copy(idx_hbm.at[wid], idx_v)

        @pl.loop(0, n_ch)
        def chunk(j):
            pltpu.async_copy(lut_hbm.at[idx_v.at[j]], buf, gsem).wait()

            @pl.loop(0, CH)
            def srow(r):
                for cc in range(D_MODEL // 16):
                    sl = pl.ds(cc * 16, 16)
                    buf[r, sl] = buf[r, sl] * SCALE

            pltpu.sync_copy(buf, out_hbm.at[pl.ds(base + j * CH, CH)])

    return emb_kernel


def kernel(x, lut):
    Bb, S = x.shape
    V, Dm = lut.shape
    B = Bb * S
    idx = x.astype(jnp.int32).reshape(NW, B // (NW * CH), CH)
    out = _make_kernel(B, V)(idx, lut)
    return out.reshape(Bb, S, Dm)


# speedup vs baseline: 1.1849x; 1.1849x over previous
import functools
import math

import jax
import jax.numpy as jnp
from jax import lax
from jax.experimental import pallas as pl
from jax.experimental.pallas import tpu as pltpu
from jax.experimental.pallas import tpu_sc as plsc

D_MODEL = 64
SCALE = math.sqrt(D_MODEL)

NW = 32
CH = 128
NB = 4
LEAD = 2


def _make_kernel(B, V):
    b_per_w = B // NW
    n_ch = b_per_w // CH

    mesh = plsc.VectorSubcoreMesh(core_axis_name="c", subcore_axis_name="s")

    @functools.partial(
        pl.kernel,
        mesh=mesh,
        out_type=jax.ShapeDtypeStruct((B, D_MODEL), jnp.float32),
        scratch_types=[
            pltpu.VMEM((n_ch, CH), jnp.int32),
            [pltpu.VMEM((CH, D_MODEL), jnp.float32) for _ in range(NB)],
            [pltpu.SemaphoreType.DMA for _ in range(NB)],
            [pltpu.SemaphoreType.DMA for _ in range(NB)],
        ],
        compiler_params=pltpu.CompilerParams(use_tc_tiling_on_sc=False),
    )
    def emb_kernel(idx_hbm, lut_hbm, out_hbm, idx_v, bufs, gsems, osems):
        wid = lax.axis_index("s") * 2 + lax.axis_index("c")
        base = wid * b_per_w
        pltpu.sync_copy(idx_hbm.at[wid], idx_v)

        for j in range(LEAD):
            pltpu.async_copy(lut_hbm.at[idx_v.at[j]], bufs[j], gsems[j])

        @pl.loop(0, n_ch, step=NB)
        def block(j0):
            for b in range(NB):
                j = j0 + b
                buf = bufs[b]
                pltpu.make_async_copy(lut_hbm.at[idx_v.at[0]], buf,
                                      gsems[b]).wait()

                @pl.loop(0, CH, unroll=4)
                def srow(r):
                    for cc in range(D_MODEL // 16):
                        sl = pl.ds(cc * 16, 16)
                        buf[r, sl] = buf[r, sl] * SCALE

                pltpu.async_copy(
                    buf, out_hbm.at[pl.ds(base + j * CH, CH)], osems[b])

                jn = j + LEAD
                bn = (b + LEAD) % NB

                @pl.when(jn < n_ch)
                def _issue():
                    @pl.when(jn >= NB)
                    def _drain():
                        pltpu.make_async_copy(
                            bufs[bn], out_hbm.at[pl.ds(base, CH)],
                            osems[bn]).wait()

                    pltpu.async_copy(
                        lut_hbm.at[idx_v.at[jn]], bufs[bn], gsems[bn])

        for j in range(n_ch - NB, n_ch):
            b = j % NB
            pltpu.make_async_copy(
                bufs[b], out_hbm.at[pl.ds(base, CH)], osems[b]).wait()

    return emb_kernel


def kernel(x, lut):
    Bb, S = x.shape
    V, Dm = lut.shape
    B = Bb * S
    idx = x.astype(jnp.int32).reshape(NW, B // (NW * CH), CH)
    out = _make_kernel(B, V)(idx, lut)
    return out.reshape(Bb, S, Dm)
